# baseline (speedup 1.0000x reference)
import functools

import jax
import jax.numpy as jnp
from jax import lax
from jax.experimental import pallas as pl
from jax.experimental.pallas import tpu as pltpu
from jax.experimental.pallas import tpu_sc as plsc


_H_BLK_SUM = 56
_H_BLK_SCALE = 32
_LANES = 16


def _colsum_body(x_ref, o_ref):
    s = jnp.sum(x_ref[...], axis=(1, 2), keepdims=True)

    @pl.when(pl.program_id(1) == 0)
    def _init():
        o_ref[...] = s

    @pl.when(pl.program_id(1) != 0)
    def _acc():
        o_ref[...] += s


def _gate_values_body(inv_hw, m_ref, w1_ref, w2t_ref, o_ref):
    y = m_ref[...] * inv_hw
    h = jax.lax.dot_general(y, w1_ref[...], (((1,), (1,)), ((), ())),
                            preferred_element_type=jnp.float32)
    h = jnp.maximum(h, 0.0)
    z = jax.lax.dot_general(h, w2t_ref[...], (((1,), (0,)), ((), ())),
                            preferred_element_type=jnp.float32)
    o_ref[...] = jax.nn.sigmoid(z)


def _make_sc_topk(b, c, kkeep):
    nv = c // _LANES
    mesh = plsc.VectorSubcoreMesh(core_axis_name="c", subcore_axis_name="s")

    @functools.partial(
        pl.kernel,
        out_type=jax.ShapeDtypeStruct((b, c), jnp.float32),
        mesh=mesh,
        scratch_types=[pltpu.VMEM((c,), jnp.float32),
                       pltpu.VMEM((c,), jnp.float32)],
    )
    def sc_topk(g_hbm, out_hbm, g_v, o_v):
        wid = lax.axis_index("s") * 2 + lax.axis_index("c")

        def vsum(vec):
            parts = [vec[j] for j in range(_LANES)]
            while len(parts) > 1:
                parts = [parts[i] + parts[i + 1]
                         for i in range(0, len(parts), 2)]
            return parts[0]

        @pl.when(wid < b)
        def _():
            pltpu.sync_copy(g_hbm.at[wid], g_v)
            vals = [g_v[pl.ds(i * _LANES, _LANES)] for i in range(nv)]
            bits = [lax.bitcast_convert_type(v, jnp.int32) for v in vals]
            zeros = jnp.zeros((_LANES,), jnp.int32)

            def tree_count(indicators):
                parts = list(indicators)
                while len(parts) > 1:
                    parts = [parts[i] + parts[i + 1] if i + 1 < len(parts)
                             else parts[i] for i in range(0, len(parts), 2)]
                return vsum(parts[0])

            def body(_, lohi):
                lo, hi = lohi
                mid = lo + ((hi - lo + 1) >> 1)
                ok = tree_count(jnp.where(bv >= mid, 1, 0)
                                for bv in bits) >= kkeep
                return (jnp.where(ok, mid, lo),
                        jnp.where(ok, hi, mid - 1))

            lo, _ = lax.fori_loop(
                0, 31, body,
                (jnp.int32(0), jnp.int32(0x3F800000)))

            need = kkeep - tree_count(jnp.where(bv > lo, 1, 0)
                                      for bv in bits)

            def body2(_, lohi):
                lo2, hi2 = lohi
                mid2 = lo2 + ((hi2 - lo2) >> 1)
                ok = tree_count(
                    jnp.where((bits[i] == lo)
                              & (lax.iota(jnp.int32, _LANES)
                                 + (i * _LANES) <= mid2), 1, 0)
                    for i in range(nv)) >= need
                return (jnp.where(ok, lo2, mid2 + 1),
                        jnp.where(ok, mid2, hi2))

            tie_hi, _ = lax.fori_loop(
                0, 9, body2, (jnp.int32(0), jnp.int32(c - 1)))

            for i in range(nv):
                bv, v = bits[i], vals[i]
                idx = lax.iota(jnp.int32, _LANES) + (i * _LANES)
                keep = (bv > lo) | ((bv == lo) & (idx <= tie_hi))
                o_v[pl.ds(i * _LANES, _LANES)] = jnp.where(keep, v, 0.0)
            pltpu.sync_copy(o_v, out_hbm.at[wid])

    return sc_topk


def _scale_body(x_ref, g_ref, o_ref):
    o_ref[...] = x_ref[...] * g_ref[...]


def kernel(x, W1, W2):
    b, c, h, w = x.shape
    hw = h * w
    kkeep = c // 2
    hb1 = _H_BLK_SUM
    hb = _H_BLK_SCALE
    xt = jnp.transpose(x, (0, 2, 3, 1))

    sums = pl.pallas_call(
        _colsum_body,
        grid=(b, h // hb1),
        in_specs=[pl.BlockSpec((1, hb1, w, c), lambda i, j: (i, j, 0, 0))],
        out_specs=pl.BlockSpec((1, 1, 1, c), lambda i, j: (i, 0, 0, 0)),
        out_shape=jax.ShapeDtypeStruct((b, 1, 1, c), jnp.float32),
    )(xt)

    gvals = pl.pallas_call(
        lambda *refs: _gate_values_body(1.0 / hw, *refs),
        in_specs=[pl.BlockSpec((b, c), lambda: (0, 0)),
                  pl.BlockSpec(W1.shape, lambda: (0, 0)),
                  pl.BlockSpec((W2.shape[1], W2.shape[0]), lambda: (0, 0))],
        out_specs=pl.BlockSpec((b, c), lambda: (0, 0)),
        out_shape=jax.ShapeDtypeStruct((b, c), jnp.float32),
    )(sums.reshape(b, c), W1, W2.T)

    gate = _make_sc_topk(b, c, kkeep)(gvals)

    out_t = pl.pallas_call(
        _scale_body,
        grid=(b, h // hb),
        in_specs=[pl.BlockSpec((1, hb, w, c), lambda i, j: (i, j, 0, 0)),
                  pl.BlockSpec((1, 1, 1, c), lambda i, j: (i, 0, 0, 0))],
        out_specs=pl.BlockSpec((1, hb, w, c), lambda i, j: (i, j, 0, 0)),
        out_shape=jax.ShapeDtypeStruct((b, h, w, c), jnp.float32),
    )(xt, gate.reshape(b, 1, 1, c))

    return jnp.transpose(out_t, (0, 3, 1, 2))


# speedup vs baseline: 1.0003x; 1.0003x over previous
import functools

import jax
import jax.numpy as jnp
from jax import lax
from jax.experimental import pallas as pl
from jax.experimental.pallas import tpu as pltpu
from jax.experimental.pallas import tpu_sc as plsc


_H_BLK_SUM = 32
_H_BLK_SCALE = 32
_LANES = 16


def _colsum_body(x_ref, o_ref):
    s = jnp.sum(x_ref[...], axis=(1, 2), keepdims=True)

    @pl.when(pl.program_id(1) == 0)
    def _init():
        o_ref[...] = s

    @pl.when(pl.program_id(1) != 0)
    def _acc():
        o_ref[...] += s


def _gate_values_body(inv_hw, m_ref, w1_ref, w2t_ref, o_ref):
    y = m_ref[...] * inv_hw
    h = jax.lax.dot_general(y, w1_ref[...], (((1,), (1,)), ((), ())),
                            preferred_element_type=jnp.float32)
    h = jnp.maximum(h, 0.0)
    z = jax.lax.dot_general(h, w2t_ref[...], (((1,), (0,)), ((), ())),
                            preferred_element_type=jnp.float32)
    o_ref[...] = jax.nn.sigmoid(z)


def _make_sc_topk(b, c, kkeep):
    nv = c // _LANES
    mesh = plsc.VectorSubcoreMesh(core_axis_name="c", subcore_axis_name="s")

    @functools.partial(
        pl.kernel,
        out_type=jax.ShapeDtypeStruct((b, c), jnp.float32),
        mesh=mesh,
        scratch_types=[pltpu.VMEM((c,), jnp.float32),
                       pltpu.VMEM((c,), jnp.float32)],
    )
    def sc_topk(g_hbm, out_hbm, g_v, o_v):
        wid = lax.axis_index("s") * 2 + lax.axis_index("c")

        def vsum(vec):
            parts = [vec[j] for j in range(_LANES)]
            while len(parts) > 1:
                parts = [parts[i] + parts[i + 1]
                         for i in range(0, len(parts), 2)]
            return parts[0]

        @pl.when(wid < b)
        def _():
            pltpu.sync_copy(g_hbm.at[wid], g_v)
            vals = [g_v[pl.ds(i * _LANES, _LANES)] for i in range(nv)]
            bits = [lax.bitcast_convert_type(v, jnp.int32) for v in vals]
            zeros = jnp.zeros((_LANES,), jnp.int32)

            def tree_count(indicators):
                parts = list(indicators)
                while len(parts) > 1:
                    parts = [parts[i] + parts[i + 1] if i + 1 < len(parts)
                             else parts[i] for i in range(0, len(parts), 2)]
                return vsum(parts[0])

            def body(_, lohi):
                lo, hi = lohi
                mid = lo + ((hi - lo + 1) >> 1)
                ok = tree_count(jnp.where(bv >= mid, 1, 0)
                                for bv in bits) >= kkeep
                return (jnp.where(ok, mid, lo),
                        jnp.where(ok, hi, mid - 1))

            lo, _ = lax.fori_loop(
                0, 31, body,
                (jnp.int32(0), jnp.int32(0x3F800000)))

            need = kkeep - tree_count(jnp.where(bv > lo, 1, 0)
                                      for bv in bits)

            def body2(_, lohi):
                lo2, hi2 = lohi
                mid2 = lo2 + ((hi2 - lo2) >> 1)
                ok = tree_count(
                    jnp.where((bits[i] == lo)
                              & (lax.iota(jnp.int32, _LANES)
                                 + (i * _LANES) <= mid2), 1, 0)
                    for i in range(nv)) >= need
                return (jnp.where(ok, lo2, mid2 + 1),
                        jnp.where(ok, mid2, hi2))

            tie_hi, _ = lax.fori_loop(
                0, 9, body2, (jnp.int32(0), jnp.int32(c - 1)))

            for i in range(nv):
                bv, v = bits[i], vals[i]
                idx = lax.iota(jnp.int32, _LANES) + (i * _LANES)
                keep = (bv > lo) | ((bv == lo) & (idx <= tie_hi))
                o_v[pl.ds(i * _LANES, _LANES)] = jnp.where(keep, v, 0.0)
            pltpu.sync_copy(o_v, out_hbm.at[wid])

    return sc_topk


def _scale_body(x_ref, g_ref, o_ref):
    o_ref[...] = x_ref[...] * g_ref[...]


def kernel(x, W1, W2):
    b, c, h, w = x.shape
    hw = h * w
    kkeep = c // 2
    hb1 = _H_BLK_SUM
    hb = _H_BLK_SCALE
    xt = jnp.transpose(x, (0, 2, 3, 1))

    sums = pl.pallas_call(
        _colsum_body,
        grid=(b, h // hb1),
        in_specs=[pl.BlockSpec((1, hb1, w, c), lambda i, j: (i, j, 0, 0))],
        out_specs=pl.BlockSpec((1, 1, 1, c), lambda i, j: (i, 0, 0, 0)),
        out_shape=jax.ShapeDtypeStruct((b, 1, 1, c), jnp.float32),
    )(xt)

    gvals = pl.pallas_call(
        lambda *refs: _gate_values_body(1.0 / hw, *refs),
        in_specs=[pl.BlockSpec((b, c), lambda: (0, 0)),
                  pl.BlockSpec(W1.shape, lambda: (0, 0)),
                  pl.BlockSpec((W2.shape[1], W2.shape[0]), lambda: (0, 0))],
        out_specs=pl.BlockSpec((b, c), lambda: (0, 0)),
        out_shape=jax.ShapeDtypeStruct((b, c), jnp.float32),
    )(sums.reshape(b, c), W1, W2.T)

    gate = _make_sc_topk(b, c, kkeep)(gvals)

    out_t = pl.pallas_call(
        _scale_body,
        grid=(b, h // hb),
        in_specs=[pl.BlockSpec((1, hb, w, c), lambda i, j: (i, j, 0, 0)),
                  pl.BlockSpec((1, 1, 1, c), lambda i, j: (i, 0, 0, 0))],
        out_specs=pl.BlockSpec((1, hb, w, c), lambda i, j: (i, j, 0, 0)),
        out_shape=jax.ShapeDtypeStruct((b, h, w, c), jnp.float32),
    )(xt, gate.reshape(b, 1, 1, c))

    return jnp.transpose(out_t, (0, 3, 1, 2))


# speedup vs baseline: 1.0021x; 1.0018x over previous
import functools

import jax
import jax.numpy as jnp
from jax import lax
from jax.experimental import pallas as pl
from jax.experimental.pallas import tpu as pltpu
from jax.experimental.pallas import tpu_sc as plsc


_H_BLK_SUM = 32
_H_BLK_SCALE = 32
_LANES = 16


def _colsum_body(x_ref, o_ref):
    s = jnp.sum(x_ref[...], axis=(1, 2), keepdims=True)

    @pl.when(pl.program_id(1) == 0)
    def _init():
        o_ref[...] = s

    @pl.when(pl.program_id(1) != 0)
    def _acc():
        o_ref[...] += s


def _gate_values_body(inv_hw, m_ref, w1_ref, w2t_ref, o_ref):
    y = m_ref[...] * inv_hw
    h = jax.lax.dot_general(y, w1_ref[...], (((1,), (1,)), ((), ())),
                            preferred_element_type=jnp.float32)
    h = jnp.maximum(h, 0.0)
    z = jax.lax.dot_general(h, w2t_ref[...], (((1,), (0,)), ((), ())),
                            preferred_element_type=jnp.float32)
    o_ref[...] = jax.nn.sigmoid(z)


def _make_sc_topk(b, c, kkeep):
    nv = c // _LANES
    mesh = plsc.VectorSubcoreMesh(core_axis_name="c", subcore_axis_name="s")

    @functools.partial(
        pl.kernel,
        out_type=jax.ShapeDtypeStruct((b, c), jnp.float32),
        mesh=mesh,
        scratch_types=[pltpu.VMEM((c,), jnp.float32),
                       pltpu.VMEM((c,), jnp.float32)],
    )
    def sc_topk(g_hbm, out_hbm, g_v, o_v):
        wid = lax.axis_index("s") * 2 + lax.axis_index("c")

        def vsum(vec):
            parts = [vec[j] for j in range(_LANES)]
            while len(parts) > 1:
                parts = [parts[i] + parts[i + 1]
                         for i in range(0, len(parts), 2)]
            return parts[0]

        @pl.when(wid < b)
        def _():
            pltpu.sync_copy(g_hbm.at[wid], g_v)
            vals = [g_v[pl.ds(i * _LANES, _LANES)] for i in range(nv)]
            bits = [lax.bitcast_convert_type(v, jnp.int32) for v in vals]

            def tree_count(indicators):
                parts = list(indicators)
                while len(parts) > 1:
                    parts = [parts[i] + parts[i + 1] if i + 1 < len(parts)
                             else parts[i] for i in range(0, len(parts), 2)]
                return vsum(parts[0])

            def body(_, lohi):
                lo, hi = lohi
                mid = lo + ((hi - lo + 1) >> 1)
                ok = tree_count(jnp.where(bv >= mid, 1, 0)
                                for bv in bits) >= kkeep
                return (jnp.where(ok, mid, lo),
                        jnp.where(ok, hi, mid - 1))

            lo, _ = lax.fori_loop(
                0, 31, body,
                (jnp.int32(0), jnp.int32(0x3F800000)))

            need = kkeep - tree_count(jnp.where(bv > lo, 1, 0)
                                      for bv in bits)

            def body2(_, lohi):
                lo2, hi2 = lohi
                mid2 = lo2 + ((hi2 - lo2) >> 1)
                ok = tree_count(
                    jnp.where((bits[i] == lo)
                              & (lax.iota(jnp.int32, _LANES)
                                 + (i * _LANES) <= mid2), 1, 0)
                    for i in range(nv)) >= need
                return (jnp.where(ok, lo2, mid2 + 1),
                        jnp.where(ok, mid2, hi2))

            tie_hi, _ = lax.fori_loop(
                0, 9, body2, (jnp.int32(0), jnp.int32(c - 1)))

            for i in range(nv):
                bv, v = bits[i], vals[i]
                idx = lax.iota(jnp.int32, _LANES) + (i * _LANES)
                keep = (bv > lo) | ((bv == lo) & (idx <= tie_hi))
                o_v[pl.ds(i * _LANES, _LANES)] = jnp.where(keep, v, 0.0)
            pltpu.sync_copy(o_v, out_hbm.at[wid])

    return sc_topk


def _scale_body(x_ref, g_ref, o_ref):
    o_ref[...] = x_ref[...] * g_ref[...]


def kernel(x, W1, W2):
    b, c, h, w = x.shape
    hw = h * w
    kkeep = c // 2
    hb1 = _H_BLK_SUM
    hb = _H_BLK_SCALE
    xt = jnp.transpose(x, (0, 2, 3, 1))

    sums = pl.pallas_call(
        _colsum_body,
        grid=(b, h // hb1),
        in_specs=[pl.BlockSpec((1, hb1, w, c), lambda i, j: (i, j, 0, 0))],
        out_specs=pl.BlockSpec((1, 1, 1, c), lambda i, j: (i, 0, 0, 0)),
        out_shape=jax.ShapeDtypeStruct((b, 1, 1, c), jnp.float32),
    )(xt)

    gvals = pl.pallas_call(
        lambda *refs: _gate_values_body(1.0 / hw, *refs),
        in_specs=[pl.BlockSpec((b, c), lambda: (0, 0)),
                  pl.BlockSpec(W1.shape, lambda: (0, 0)),
                  pl.BlockSpec((W2.shape[1], W2.shape[0]), lambda: (0, 0))],
        out_specs=pl.BlockSpec((b, c), lambda: (0, 0)),
        out_shape=jax.ShapeDtypeStruct((b, c), jnp.float32),
    )(sums.reshape(b, c), W1, W2.T)

    gate = _make_sc_topk(b, c, kkeep)(gvals)

    out_t = pl.pallas_call(
        _scale_body,
        grid=(b, h // hb),
        in_specs=[pl.BlockSpec((1, hb, w, c), lambda i, j: (i, j, 0, 0)),
                  pl.BlockSpec((1, 1, 1, c), lambda i, j: (i, 0, 0, 0))],
        out_specs=pl.BlockSpec((1, hb, w, c), lambda i, j: (i, j, 0, 0)),
        out_shape=jax.ShapeDtypeStruct((b, h, w, c), jnp.float32),
    )(xt, gate.reshape(b, 1, 1, c))

    return jnp.transpose(out_t, (0, 3, 1, 2))
